# Initial kernel scaffold; baseline (speedup 1.0000x reference)
import jax
import jax.numpy as jnp
from jax.experimental import pallas as pl


def kernel(x, edge_index, edge_attr, a, W_enc, emb1, emb2, W1, b1, W2, b2):
    raise NotImplementedError("write your pallas kernel here")


# speedup vs baseline: 4.9821x; 4.9821x over previous
import functools

import jax
import jax.numpy as jnp
from jax import lax
from jax.experimental import pallas as pl
from jax.experimental.pallas import tpu as pltpu
from jax.experimental.pallas import tpu_sc as plsc

N_NODES = 10000
HID = 128
OUT = 128
N_EDGES = 320000

NC = 2
NS = 16
NW = NC * NS

CH = 128
EPT = 10240
E_PAD = NW * EPT
CHUNKS = EPT // CH

ACC_ROWS = 10112
ROWS_PER_TILE = ACC_ROWS // NS
NBT = 6
NCODE = NBT * NBT
HB_PER_TILE = 23552
HBINS = NS * HB_PER_TILE

_mesh = plsc.VectorSubcoreMesh(core_axis_name="c", subcore_axis_name="s")


@functools.partial(
    pl.kernel,
    out_type=[
        jax.ShapeDtypeStruct((NC, ACC_ROWS, HID), jnp.float32),
        jax.ShapeDtypeStruct((NC * HBINS,), jnp.float32),
    ],
    mesh=_mesh,
    scratch_types=[
        pltpu.VMEM((CH,), jnp.int32),
        pltpu.VMEM((CH,), jnp.int32),
        pltpu.VMEM((CH,), jnp.int32),
        pltpu.VMEM((CH, HID), jnp.float32),
        pltpu.VMEM((CH,), jnp.float32),
        pltpu.VMEM_SHARED((ACC_ROWS, HID), jnp.float32),
        pltpu.VMEM_SHARED((HBINS,), jnp.float32),
        pltpu.SemaphoreType.DMA,
    ],
)
def _sc_aggregate(h_hbm, src_hbm, dst_hbm, hix_hbm, z2d_hbm, z1d_hbm,
                  acc_out, hist_out,
                  src_v, dst_v, hix_v, rows_v, ones_v, acc_sh, hist_sh, sem):
    cid = lax.axis_index("c")
    sid = lax.axis_index("s")
    wid = sid * NC + cid

    pltpu.sync_copy(z2d_hbm, acc_sh.at[pl.ds(sid * ROWS_PER_TILE, ROWS_PER_TILE)])
    pltpu.sync_copy(z1d_hbm, hist_sh.at[pl.ds(sid * HB_PER_TILE, HB_PER_TILE)])
    for j in range(CH // 16):
        ones_v[pl.ds(j * 16, 16)] = jnp.ones((16,), jnp.float32)
    plsc.subcore_barrier()

    base = wid * EPT

    def chunk(i, carry):
        off = base + i * CH
        pltpu.sync_copy(src_hbm.at[pl.ds(off, CH)], src_v)
        pltpu.sync_copy(dst_hbm.at[pl.ds(off, CH)], dst_v)
        pltpu.sync_copy(hix_hbm.at[pl.ds(off, CH)], hix_v)
        pltpu.async_copy(h_hbm.at[src_v], rows_v, sem).wait()
        pltpu.sync_copy(rows_v, acc_sh.at[dst_v], add=True)
        pltpu.sync_copy(ones_v, hist_sh.at[hix_v], add=True)
        return carry

    lax.fori_loop(0, CHUNKS, chunk, 0)
    plsc.subcore_barrier()

    pltpu.sync_copy(acc_sh.at[pl.ds(sid * ROWS_PER_TILE, ROWS_PER_TILE)],
                    acc_out.at[cid, pl.ds(sid * ROWS_PER_TILE, ROWS_PER_TILE)])
    pltpu.sync_copy(hist_sh.at[pl.ds(sid * HB_PER_TILE, HB_PER_TILE)],
                    hist_out.at[pl.ds(cid * HBINS + sid * HB_PER_TILE, HB_PER_TILE)])


def _tc_encode(x_ref, w_ref, a_ref, o_ref):
    xb = x_ref[...]
    h = jnp.maximum(xb, 0.0) + a_ref[0, 0] * jnp.minimum(xb, 0.0)
    o_ref[...] = lax.dot_general(h, w_ref[...], (((1,), (1,)), ((), ())),
                                 preferred_element_type=jnp.float32)


def _tc_mlp(acc0_ref, acc1_ref, h_ref, hi0_ref, hi1_ref, e_ref, sl_ref,
            w1_ref, b1_ref, w2_ref, b2_ref, o_ref):
    aggr = acc0_ref[...] + acc1_ref[...] + h_ref[...] + sl_ref[...]
    aggr = aggr + jnp.dot(hi0_ref[...] + hi1_ref[...], e_ref[...],
                          preferred_element_type=jnp.float32)
    hid = lax.dot_general(aggr, w1_ref[...], (((1,), (1,)), ((), ())),
                          preferred_element_type=jnp.float32) + b1_ref[...]
    hid = hid * jax.nn.sigmoid(hid)
    o_ref[...] = lax.dot_general(hid, w2_ref[...], (((1,), (1,)), ((), ())),
                                 preferred_element_type=jnp.float32) + b2_ref[...]


_BLK = 1000
_GRID = N_NODES // _BLK


def kernel(x, edge_index, edge_attr, a, W_enc, emb1, emb2, W1, b1, W2, b2):
    f32 = jnp.float32
    h = pl.pallas_call(
        _tc_encode,
        grid=(_GRID,),
        in_specs=[
            pl.BlockSpec((_BLK, HID), lambda i: (i, 0)),
            pl.BlockSpec((HID, HID), lambda i: (0, 0)),
            pl.BlockSpec((1, 1), lambda i: (0, 0)),
        ],
        out_specs=pl.BlockSpec((_BLK, HID), lambda i: (i, 0)),
        out_shape=jax.ShapeDtypeStruct((N_NODES, HID), f32),
    )(x, W_enc, jnp.reshape(a, (1, 1)))

    src = edge_index[0].astype(jnp.int32)
    dst = edge_index[1].astype(jnp.int32)
    code = (edge_attr[:, 0] * NBT + edge_attr[:, 1]).astype(jnp.int32)
    pad = E_PAD - N_EDGES
    src_p = jnp.concatenate([src, jnp.zeros((pad,), jnp.int32)])
    dst_p = jnp.concatenate([dst, jnp.full((pad,), N_NODES, jnp.int32)])
    hix_p = jnp.concatenate([dst * NCODE + code,
                             jnp.full((pad,), N_NODES * NCODE, jnp.int32)])
    z2d = jnp.zeros((ROWS_PER_TILE, HID), f32)
    z1d = jnp.zeros((HB_PER_TILE,), f32)

    accs, hists = _sc_aggregate(h, src_p, dst_p, hix_p, z2d, z1d)
    accs = accs[:, :N_NODES]

    etab = (emb1[:, None, :] + emb2[None, :, :]).reshape(NCODE, HID)
    slvec = (emb1[5] + emb2[0]).reshape(1, HID)
    hi0 = hists[:N_NODES * NCODE].reshape(N_NODES, NCODE)
    hi1 = hists[HBINS:HBINS + N_NODES * NCODE].reshape(N_NODES, NCODE)
    out = pl.pallas_call(
        _tc_mlp,
        grid=(_GRID,),
        in_specs=[
            pl.BlockSpec((_BLK, HID), lambda i: (i, 0)),
            pl.BlockSpec((_BLK, HID), lambda i: (i, 0)),
            pl.BlockSpec((_BLK, HID), lambda i: (i, 0)),
            pl.BlockSpec((_BLK, NCODE), lambda i: (i, 0)),
            pl.BlockSpec((_BLK, NCODE), lambda i: (i, 0)),
            pl.BlockSpec((NCODE, HID), lambda i: (0, 0)),
            pl.BlockSpec((1, HID), lambda i: (0, 0)),
            pl.BlockSpec((2 * HID, HID), lambda i: (0, 0)),
            pl.BlockSpec((1, 2 * HID), lambda i: (0, 0)),
            pl.BlockSpec((OUT, 2 * HID), lambda i: (0, 0)),
            pl.BlockSpec((1, OUT), lambda i: (0, 0)),
        ],
        out_specs=pl.BlockSpec((_BLK, OUT), lambda i: (i, 0)),
        out_shape=jax.ShapeDtypeStruct((N_NODES, OUT), f32),
    )(accs[0], accs[1], h, hi0, hi1, etab, slvec,
      W1, jnp.reshape(b1, (1, 2 * HID)), W2, jnp.reshape(b2, (1, OUT)))
    return out


# speedup vs baseline: 5.3714x; 1.0781x over previous
import functools

import jax
import jax.numpy as jnp
from jax import lax
from jax.experimental import pallas as pl
from jax.experimental.pallas import tpu as pltpu
from jax.experimental.pallas import tpu_sc as plsc

N_NODES = 10000
HID = 128
OUT = 128
N_EDGES = 320000

NC = 2
NS = 16
NW = NC * NS

CH = 96
CHUNKS = 107
EPT = CH * CHUNKS
E_PAD = NW * EPT

ACC_ROWS = 10112
ROWS_PER_TILE = ACC_ROWS // NS
NBT = 6
NCODE = NBT * NBT
HB_PER_TILE = 22528
HBINS = NS * HB_PER_TILE

_mesh = plsc.VectorSubcoreMesh(core_axis_name="c", subcore_axis_name="s")


@functools.partial(
    pl.kernel,
    out_type=[
        jax.ShapeDtypeStruct((NC, ACC_ROWS, HID), jnp.float32),
        jax.ShapeDtypeStruct((NC * HBINS,), jnp.float32),
    ],
    mesh=_mesh,
    scratch_types=[
        pltpu.VMEM((3, CH), jnp.int32),
        pltpu.VMEM((3, CH), jnp.int32),
        pltpu.VMEM((CH, HID), jnp.float32),
        pltpu.VMEM((CH, HID), jnp.float32),
        pltpu.VMEM((CH,), jnp.float32),
        pltpu.VMEM_SHARED((ACC_ROWS, HID), jnp.float32),
        pltpu.VMEM_SHARED((HBINS,), jnp.float32),
        pltpu.SemaphoreType.DMA,
        pltpu.SemaphoreType.DMA,
        pltpu.SemaphoreType.DMA,
        pltpu.SemaphoreType.DMA,
    ],
)
def _sc_aggregate(h_hbm, idx_hbm, z2d_hbm, z1d_hbm,
                  acc_out, hist_out,
                  ib0, ib1, rb0, rb1, ones_v, acc_sh, hist_sh,
                  sg0, sg1, si0, si1):
    cid = lax.axis_index("c")
    sid = lax.axis_index("s")
    wid = sid * NC + cid

    pltpu.sync_copy(z2d_hbm, acc_sh.at[pl.ds(sid * ROWS_PER_TILE, ROWS_PER_TILE)])
    pltpu.sync_copy(z1d_hbm, hist_sh.at[pl.ds(sid * HB_PER_TILE, HB_PER_TILE)])
    for j in range(CH // 16):
        ones_v[pl.ds(j * 16, 16)] = jnp.ones((16,), jnp.float32)
    plsc.subcore_barrier()

    ibs, rbs, sgs, sis = (ib0, ib1), (rb0, rb1), (sg0, sg1), (si0, si1)

    pltpu.sync_copy(idx_hbm.at[wid, 0], ib0)
    pltpu.async_copy(h_hbm.at[ib0.at[0]], rb0, sg0)
    pltpu.async_copy(idx_hbm.at[wid, 1], ib1, si1)

    def step(i, b, last):
        nb = 1 - b
        if not last:
            pltpu.make_async_copy(idx_hbm.at[wid, i + 1], ibs[nb], sis[nb]).wait()
            pltpu.async_copy(h_hbm.at[ibs[nb].at[0]], rbs[nb], sgs[nb])
        pltpu.make_async_copy(h_hbm.at[ibs[b].at[0]], rbs[b], sgs[b]).wait()
        pltpu.sync_copy(rbs[b], acc_sh.at[ibs[b].at[1]], add=True)
        pltpu.sync_copy(ones_v, hist_sh.at[ibs[b].at[2]], add=True)
        if not last:
            nxt = jnp.minimum(i + 2, CHUNKS - 1)
            pltpu.async_copy(idx_hbm.at[wid, nxt], ibs[b], sis[b])

    def outer(g, carry):
        for b in range(2):
            step(2 * g + b, b, last=False)
        return carry

    lax.fori_loop(0, (CHUNKS - 1) // 2, outer, 0)
    pltpu.make_async_copy(idx_hbm.at[wid, CHUNKS - 1], ib1, si1).wait()
    step(CHUNKS - 1, (CHUNKS - 1) % 2, last=True)
    plsc.subcore_barrier()

    pltpu.sync_copy(acc_sh.at[pl.ds(sid * ROWS_PER_TILE, ROWS_PER_TILE)],
                    acc_out.at[cid, pl.ds(sid * ROWS_PER_TILE, ROWS_PER_TILE)])
    pltpu.sync_copy(hist_sh.at[pl.ds(sid * HB_PER_TILE, HB_PER_TILE)],
                    hist_out.at[pl.ds(cid * HBINS + sid * HB_PER_TILE, HB_PER_TILE)])


def _tc_encode(x_ref, w_ref, a_ref, o_ref):
    xb = x_ref[...]
    h = jnp.maximum(xb, 0.0) + a_ref[0, 0] * jnp.minimum(xb, 0.0)
    o_ref[...] = lax.dot_general(h, w_ref[...], (((1,), (1,)), ((), ())),
                                 preferred_element_type=jnp.float32)


def _tc_mlp(acc0_ref, acc1_ref, h_ref, hi0_ref, hi1_ref, e_ref, sl_ref,
            w1_ref, b1_ref, w2_ref, b2_ref, o_ref):
    aggr = acc0_ref[...] + acc1_ref[...] + h_ref[...] + sl_ref[...]
    aggr = aggr + jnp.dot(hi0_ref[...] + hi1_ref[...], e_ref[...],
                          preferred_element_type=jnp.float32)
    hid = lax.dot_general(aggr, w1_ref[...], (((1,), (1,)), ((), ())),
                          preferred_element_type=jnp.float32) + b1_ref[...]
    hid = hid * jax.nn.sigmoid(hid)
    o_ref[...] = lax.dot_general(hid, w2_ref[...], (((1,), (1,)), ((), ())),
                                 preferred_element_type=jnp.float32) + b2_ref[...]


_BLK = 1000
_GRID = N_NODES // _BLK


def kernel(x, edge_index, edge_attr, a, W_enc, emb1, emb2, W1, b1, W2, b2):
    f32 = jnp.float32
    h = pl.pallas_call(
        _tc_encode,
        grid=(_GRID,),
        in_specs=[
            pl.BlockSpec((_BLK, HID), lambda i: (i, 0)),
            pl.BlockSpec((HID, HID), lambda i: (0, 0)),
            pl.BlockSpec((1, 1), lambda i: (0, 0)),
        ],
        out_specs=pl.BlockSpec((_BLK, HID), lambda i: (i, 0)),
        out_shape=jax.ShapeDtypeStruct((N_NODES, HID), f32),
    )(x, W_enc, jnp.reshape(a, (1, 1)))

    src = edge_index[0].astype(jnp.int32)
    dst = edge_index[1].astype(jnp.int32)
    code = (edge_attr[:, 0] * NBT + edge_attr[:, 1]).astype(jnp.int32)
    pad = E_PAD - N_EDGES
    src_p = jnp.concatenate([src, jnp.zeros((pad,), jnp.int32)]
                            ).reshape(NW, CHUNKS, CH)
    dst_p = jnp.concatenate([dst, jnp.full((pad,), N_NODES, jnp.int32)]
                            ).reshape(NW, CHUNKS, CH)
    hix_p = jnp.concatenate([dst * NCODE + code,
                             jnp.full((pad,), N_NODES * NCODE, jnp.int32)]
                            ).reshape(NW, CHUNKS, CH)
    idx_p = jnp.stack([src_p, dst_p, hix_p], axis=2)
    z2d = jnp.zeros((ROWS_PER_TILE, HID), f32)
    z1d = jnp.zeros((HB_PER_TILE,), f32)

    accs, hists = _sc_aggregate(h, idx_p, z2d, z1d)
    accs = accs[:, :N_NODES]

    etab = (emb1[:, None, :] + emb2[None, :, :]).reshape(NCODE, HID)
    slvec = (emb1[5] + emb2[0]).reshape(1, HID)
    hi0 = hists[:N_NODES * NCODE].reshape(N_NODES, NCODE)
    hi1 = hists[HBINS:HBINS + N_NODES * NCODE].reshape(N_NODES, NCODE)
    out = pl.pallas_call(
        _tc_mlp,
        grid=(_GRID,),
        in_specs=[
            pl.BlockSpec((_BLK, HID), lambda i: (i, 0)),
            pl.BlockSpec((_BLK, HID), lambda i: (i, 0)),
            pl.BlockSpec((_BLK, HID), lambda i: (i, 0)),
            pl.BlockSpec((_BLK, NCODE), lambda i: (i, 0)),
            pl.BlockSpec((_BLK, NCODE), lambda i: (i, 0)),
            pl.BlockSpec((NCODE, HID), lambda i: (0, 0)),
            pl.BlockSpec((1, HID), lambda i: (0, 0)),
            pl.BlockSpec((2 * HID, HID), lambda i: (0, 0)),
            pl.BlockSpec((1, 2 * HID), lambda i: (0, 0)),
            pl.BlockSpec((OUT, 2 * HID), lambda i: (0, 0)),
            pl.BlockSpec((1, OUT), lambda i: (0, 0)),
        ],
        out_specs=pl.BlockSpec((_BLK, OUT), lambda i: (i, 0)),
        out_shape=jax.ShapeDtypeStruct((N_NODES, OUT), f32),
    )(accs[0], accs[1], h, hi0, hi1, etab, slvec,
      W1, jnp.reshape(b1, (1, 2 * HID)), W2, jnp.reshape(b2, (1, OUT)))
    return out


# speedup vs baseline: 5.8191x; 1.0833x over previous
import functools

import jax
import jax.numpy as jnp
from jax import lax
from jax.experimental import pallas as pl
from jax.experimental.pallas import tpu as pltpu
from jax.experimental.pallas import tpu_sc as plsc

N_NODES = 10000
HID = 128
OUT = 128
N_EDGES = 320000

NC = 2
NS = 16
NW = NC * NS

CH = 96
CHUNKS0 = 171
CHUNKS1 = 43
TOTAL_CHUNKS = NS * (CHUNKS0 + CHUNKS1)
C1_BASE = NS * CHUNKS0
E_PAD = TOTAL_CHUNKS * CH

ACC_ROWS = 10112
ROWS_PER_TILE = ACC_ROWS // NS
NBT = 6
NCODE = NBT * NBT
HB_PER_TILE = 22528
HBINS = NS * HB_PER_TILE

_mesh = plsc.VectorSubcoreMesh(core_axis_name="c", subcore_axis_name="s")


@functools.partial(
    pl.kernel,
    out_type=[
        jax.ShapeDtypeStruct((NC, ACC_ROWS, HID), jnp.float32),
        jax.ShapeDtypeStruct((NC * HBINS,), jnp.float32),
    ],
    mesh=_mesh,
    scratch_types=[
        pltpu.VMEM((3, CH), jnp.int32),
        pltpu.VMEM((3, CH), jnp.int32),
        pltpu.VMEM((CH, HID), jnp.float32),
        pltpu.VMEM((CH, HID), jnp.float32),
        pltpu.VMEM((CH,), jnp.float32),
        pltpu.VMEM_SHARED((ACC_ROWS, HID), jnp.float32),
        pltpu.VMEM_SHARED((HBINS,), jnp.float32),
        pltpu.SemaphoreType.DMA,
        pltpu.SemaphoreType.DMA,
        pltpu.SemaphoreType.DMA,
        pltpu.SemaphoreType.DMA,
    ],
)
def _sc_aggregate(h_hbm, idx_hbm, z2d_hbm, z1d_hbm,
                  acc_out, hist_out,
                  ib0, ib1, rb0, rb1, ones_v, acc_sh, hist_sh,
                  sg0, sg1, si0, si1):
    cid = lax.axis_index("c")
    sid = lax.axis_index("s")
    nch = jnp.where(cid == 0, CHUNKS0, CHUNKS1)
    cbase = jnp.where(cid == 0, sid * CHUNKS0, C1_BASE + sid * CHUNKS1)

    pltpu.sync_copy(z2d_hbm, acc_sh.at[pl.ds(sid * ROWS_PER_TILE, ROWS_PER_TILE)])
    pltpu.sync_copy(z1d_hbm, hist_sh.at[pl.ds(sid * HB_PER_TILE, HB_PER_TILE)])
    for j in range(CH // 16):
        ones_v[pl.ds(j * 16, 16)] = jnp.ones((16,), jnp.float32)
    plsc.subcore_barrier()

    ibs, rbs, sgs, sis = (ib0, ib1), (rb0, rb1), (sg0, sg1), (si0, si1)

    pltpu.sync_copy(idx_hbm.at[cbase], ib0)
    pltpu.async_copy(h_hbm.at[ib0.at[0]], rb0, sg0)
    pltpu.async_copy(idx_hbm.at[cbase + 1], ib1, si1)

    def step(i, b, last):
        nb = 1 - b
        if not last:
            pltpu.make_async_copy(idx_hbm.at[cbase + i + 1], ibs[nb], sis[nb]).wait()
            pltpu.async_copy(h_hbm.at[ibs[nb].at[0]], rbs[nb], sgs[nb])
        pltpu.make_async_copy(h_hbm.at[ibs[b].at[0]], rbs[b], sgs[b]).wait()
        pltpu.sync_copy(rbs[b], acc_sh.at[ibs[b].at[1]], add=True)
        pltpu.sync_copy(ones_v, hist_sh.at[ibs[b].at[2]], add=True)
        if not last:
            nxt = jnp.minimum(i + 2, nch - 1)
            pltpu.async_copy(idx_hbm.at[cbase + nxt], ibs[b], sis[b])

    def outer(g, carry):
        for b in range(2):
            step(2 * g + b, b, last=False)
        return carry

    lax.fori_loop(0, (nch - 1) // 2, outer, 0)
    pltpu.make_async_copy(idx_hbm.at[cbase + nch - 1], ib1, si1).wait()
    step(nch - 1, 0, last=True)
    plsc.subcore_barrier()

    pltpu.sync_copy(acc_sh.at[pl.ds(sid * ROWS_PER_TILE, ROWS_PER_TILE)],
                    acc_out.at[cid, pl.ds(sid * ROWS_PER_TILE, ROWS_PER_TILE)])
    pltpu.sync_copy(hist_sh.at[pl.ds(sid * HB_PER_TILE, HB_PER_TILE)],
                    hist_out.at[pl.ds(cid * HBINS + sid * HB_PER_TILE, HB_PER_TILE)])


def _tc_encode(x_ref, w_ref, a_ref, o_ref):
    xb = x_ref[...]
    h = jnp.maximum(xb, 0.0) + a_ref[0, 0] * jnp.minimum(xb, 0.0)
    o_ref[...] = lax.dot_general(h, w_ref[...], (((1,), (1,)), ((), ())),
                                 preferred_element_type=jnp.float32)


def _tc_mlp(acc0_ref, acc1_ref, h_ref, hi0_ref, hi1_ref, e_ref, sl_ref,
            w1_ref, b1_ref, w2_ref, b2_ref, o_ref):
    aggr = acc0_ref[...] + acc1_ref[...] + h_ref[...] + sl_ref[...]
    aggr = aggr + jnp.dot(hi0_ref[...] + hi1_ref[...], e_ref[...],
                          preferred_element_type=jnp.float32)
    hid = lax.dot_general(aggr, w1_ref[...], (((1,), (1,)), ((), ())),
                          preferred_element_type=jnp.float32) + b1_ref[...]
    hid = hid * jax.nn.sigmoid(hid)
    o_ref[...] = lax.dot_general(hid, w2_ref[...], (((1,), (1,)), ((), ())),
                                 preferred_element_type=jnp.float32) + b2_ref[...]


_BLK = 1000
_GRID = N_NODES // _BLK


def kernel(x, edge_index, edge_attr, a, W_enc, emb1, emb2, W1, b1, W2, b2):
    f32 = jnp.float32
    h = pl.pallas_call(
        _tc_encode,
        grid=(_GRID,),
        in_specs=[
            pl.BlockSpec((_BLK, HID), lambda i: (i, 0)),
            pl.BlockSpec((HID, HID), lambda i: (0, 0)),
            pl.BlockSpec((1, 1), lambda i: (0, 0)),
        ],
        out_specs=pl.BlockSpec((_BLK, HID), lambda i: (i, 0)),
        out_shape=jax.ShapeDtypeStruct((N_NODES, HID), f32),
    )(x, W_enc, jnp.reshape(a, (1, 1)))

    src = edge_index[0].astype(jnp.int32)
    dst = edge_index[1].astype(jnp.int32)
    code = (edge_attr[:, 0] * NBT + edge_attr[:, 1]).astype(jnp.int32)
    pad = E_PAD - N_EDGES
    src_p = jnp.concatenate([src, jnp.zeros((pad,), jnp.int32)]
                            ).reshape(TOTAL_CHUNKS, CH)
    dst_p = jnp.concatenate([dst, jnp.full((pad,), N_NODES, jnp.int32)]
                            ).reshape(TOTAL_CHUNKS, CH)
    hix_p = jnp.concatenate([dst * NCODE + code,
                             jnp.full((pad,), N_NODES * NCODE, jnp.int32)]
                            ).reshape(TOTAL_CHUNKS, CH)
    idx_p = jnp.stack([src_p, dst_p, hix_p], axis=1)
    z2d = jnp.zeros((ROWS_PER_TILE, HID), f32)
    z1d = jnp.zeros((HB_PER_TILE,), f32)

    accs, hists = _sc_aggregate(h, idx_p, z2d, z1d)
    accs = accs[:, :N_NODES]

    etab = (emb1[:, None, :] + emb2[None, :, :]).reshape(NCODE, HID)
    slvec = (emb1[5] + emb2[0]).reshape(1, HID)
    hi0 = hists[:N_NODES * NCODE].reshape(N_NODES, NCODE)
    hi1 = hists[HBINS:HBINS + N_NODES * NCODE].reshape(N_NODES, NCODE)
    out = pl.pallas_call(
        _tc_mlp,
        grid=(_GRID,),
        in_specs=[
            pl.BlockSpec((_BLK, HID), lambda i: (i, 0)),
            pl.BlockSpec((_BLK, HID), lambda i: (i, 0)),
            pl.BlockSpec((_BLK, HID), lambda i: (i, 0)),
            pl.BlockSpec((_BLK, NCODE), lambda i: (i, 0)),
            pl.BlockSpec((_BLK, NCODE), lambda i: (i, 0)),
            pl.BlockSpec((NCODE, HID), lambda i: (0, 0)),
            pl.BlockSpec((1, HID), lambda i: (0, 0)),
            pl.BlockSpec((2 * HID, HID), lambda i: (0, 0)),
            pl.BlockSpec((1, 2 * HID), lambda i: (0, 0)),
            pl.BlockSpec((OUT, 2 * HID), lambda i: (0, 0)),
            pl.BlockSpec((1, OUT), lambda i: (0, 0)),
        ],
        out_specs=pl.BlockSpec((_BLK, OUT), lambda i: (i, 0)),
        out_shape=jax.ShapeDtypeStruct((N_NODES, OUT), f32),
    )(accs[0], accs[1], h, hi0, hi1, etab, slvec,
      W1, jnp.reshape(b1, (1, 2 * HID)), W2, jnp.reshape(b2, (1, OUT)))
    return out


# speedup vs baseline: 5.8624x; 1.0074x over previous
import functools

import jax
import jax.numpy as jnp
from jax import lax
from jax.experimental import pallas as pl
from jax.experimental.pallas import tpu as pltpu
from jax.experimental.pallas import tpu_sc as plsc

N_NODES = 10000
HID = 128
OUT = 128
N_EDGES = 320000

NC = 2
NS = 16
NW = NC * NS

CH = 96
CHUNKS0 = 171
CHUNKS1 = 43
TOTAL_CHUNKS = NS * (CHUNKS0 + CHUNKS1)
C1_BASE = NS * CHUNKS0
E_PAD = TOTAL_CHUNKS * CH

ACC_ROWS = 10112
ROWS_PER_TILE = ACC_ROWS // NS
NBT = 6
NCODE = NBT * NBT
HB_PER_TILE = 22528
HBINS = NS * HB_PER_TILE
ZB = 1024

_mesh = plsc.VectorSubcoreMesh(core_axis_name="c", subcore_axis_name="s")


@functools.partial(
    pl.kernel,
    out_type=[
        jax.ShapeDtypeStruct((NC, ACC_ROWS, HID), jnp.float32),
        jax.ShapeDtypeStruct((NC * HBINS,), jnp.float32),
    ],
    mesh=_mesh,
    scratch_types=[
        pltpu.VMEM((3, CH), jnp.int32),
        pltpu.VMEM((3, CH), jnp.int32),
        pltpu.VMEM((CH, HID), jnp.float32),
        pltpu.VMEM((CH, HID), jnp.float32),
        pltpu.VMEM((CH,), jnp.float32),
        pltpu.VMEM((ZB,), jnp.float32),
        pltpu.VMEM_SHARED((ACC_ROWS, HID), jnp.float32),
        pltpu.VMEM_SHARED((HBINS,), jnp.float32),
        pltpu.SemaphoreType.DMA,
        pltpu.SemaphoreType.DMA,
        pltpu.SemaphoreType.DMA,
        pltpu.SemaphoreType.DMA,
    ],
)
def _sc_aggregate(h_hbm, idx_hbm,
                  acc_out, hist_out,
                  ib0, ib1, rb0, rb1, ones_v, zb_v, acc_sh, hist_sh,
                  sg0, sg1, si0, si1):
    cid = lax.axis_index("c")
    sid = lax.axis_index("s")
    nch = jnp.where(cid == 0, CHUNKS0, CHUNKS1)
    cbase = jnp.where(cid == 0, sid * CHUNKS0, C1_BASE + sid * CHUNKS1)

    def zfill(r, carry):
        for k in range(HID // 16):
            rb0[r, pl.ds(k * 16, 16)] = jnp.zeros((16,), jnp.float32)
        return carry

    lax.fori_loop(0, CH, zfill, 0)

    def zfill1(j, carry):
        zb_v[pl.ds(j * 16, 16)] = jnp.zeros((16,), jnp.float32)
        return carry

    lax.fori_loop(0, ZB // 16, zfill1, 0)
    for lo in range(0, ROWS_PER_TILE, CH):
        n = min(CH, ROWS_PER_TILE - lo)
        pltpu.sync_copy(rb0.at[pl.ds(0, n)],
                        acc_sh.at[pl.ds(sid * ROWS_PER_TILE + lo, n)])
    for lo in range(0, HB_PER_TILE, ZB):
        n = min(ZB, HB_PER_TILE - lo)
        pltpu.sync_copy(zb_v.at[pl.ds(0, n)],
                        hist_sh.at[pl.ds(sid * HB_PER_TILE + lo, n)])
    for j in range(CH // 16):
        ones_v[pl.ds(j * 16, 16)] = jnp.ones((16,), jnp.float32)
    plsc.subcore_barrier()

    ibs, rbs, sgs, sis = (ib0, ib1), (rb0, rb1), (sg0, sg1), (si0, si1)

    pltpu.sync_copy(idx_hbm.at[cbase], ib0)
    pltpu.async_copy(h_hbm.at[ib0.at[0]], rb0, sg0)
    pltpu.async_copy(idx_hbm.at[cbase + 1], ib1, si1)

    def step(i, b, last):
        nb = 1 - b
        if not last:
            pltpu.make_async_copy(idx_hbm.at[cbase + i + 1], ibs[nb], sis[nb]).wait()
            pltpu.async_copy(h_hbm.at[ibs[nb].at[0]], rbs[nb], sgs[nb])
        pltpu.make_async_copy(h_hbm.at[ibs[b].at[0]], rbs[b], sgs[b]).wait()
        pltpu.sync_copy(rbs[b], acc_sh.at[ibs[b].at[1]], add=True)
        pltpu.sync_copy(ones_v, hist_sh.at[ibs[b].at[2]], add=True)
        if not last:
            nxt = jnp.minimum(i + 2, nch - 1)
            pltpu.async_copy(idx_hbm.at[cbase + nxt], ibs[b], sis[b])

    def outer(g, carry):
        for b in range(2):
            step(2 * g + b, b, last=False)
        return carry

    lax.fori_loop(0, (nch - 1) // 2, outer, 0)
    pltpu.make_async_copy(idx_hbm.at[cbase + nch - 1], ib1, si1).wait()
    step(nch - 1, 0, last=True)
    plsc.subcore_barrier()

    pltpu.sync_copy(acc_sh.at[pl.ds(sid * ROWS_PER_TILE, ROWS_PER_TILE)],
                    acc_out.at[cid, pl.ds(sid * ROWS_PER_TILE, ROWS_PER_TILE)])
    pltpu.sync_copy(hist_sh.at[pl.ds(sid * HB_PER_TILE, HB_PER_TILE)],
                    hist_out.at[pl.ds(cid * HBINS + sid * HB_PER_TILE, HB_PER_TILE)])


def _tc_encode(x_ref, w_ref, a_ref, o_ref):
    xb = x_ref[...]
    h = jnp.maximum(xb, 0.0) + a_ref[0, 0] * jnp.minimum(xb, 0.0)
    o_ref[...] = lax.dot_general(h, w_ref[...], (((1,), (1,)), ((), ())),
                                 preferred_element_type=jnp.float32)


def _tc_mlp(acc0_ref, acc1_ref, h_ref, hi0_ref, hi1_ref, e_ref, sl_ref,
            w1_ref, b1_ref, w2_ref, b2_ref, o_ref):
    aggr = acc0_ref[...] + acc1_ref[...] + h_ref[...] + sl_ref[...]
    aggr = aggr + jnp.dot(hi0_ref[...] + hi1_ref[...], e_ref[...],
                          preferred_element_type=jnp.float32)
    hid = lax.dot_general(aggr, w1_ref[...], (((1,), (1,)), ((), ())),
                          preferred_element_type=jnp.float32) + b1_ref[...]
    hid = hid * jax.nn.sigmoid(hid)
    o_ref[...] = lax.dot_general(hid, w2_ref[...], (((1,), (1,)), ((), ())),
                                 preferred_element_type=jnp.float32) + b2_ref[...]


_BLK = 1000
_GRID = N_NODES // _BLK


def kernel(x, edge_index, edge_attr, a, W_enc, emb1, emb2, W1, b1, W2, b2):
    f32 = jnp.float32
    h = pl.pallas_call(
        _tc_encode,
        grid=(_GRID,),
        in_specs=[
            pl.BlockSpec((_BLK, HID), lambda i: (i, 0)),
            pl.BlockSpec((HID, HID), lambda i: (0, 0)),
            pl.BlockSpec((1, 1), lambda i: (0, 0)),
        ],
        out_specs=pl.BlockSpec((_BLK, HID), lambda i: (i, 0)),
        out_shape=jax.ShapeDtypeStruct((N_NODES, HID), f32),
    )(x, W_enc, jnp.reshape(a, (1, 1)))

    src = edge_index[0].astype(jnp.int32)
    dst = edge_index[1].astype(jnp.int32)
    code = (edge_attr[:, 0] * NBT + edge_attr[:, 1]).astype(jnp.int32)
    pad = E_PAD - N_EDGES
    src_p = jnp.concatenate([src, jnp.zeros((pad,), jnp.int32)]
                            ).reshape(TOTAL_CHUNKS, CH)
    dst_p = jnp.concatenate([dst, jnp.full((pad,), N_NODES, jnp.int32)]
                            ).reshape(TOTAL_CHUNKS, CH)
    hix_p = jnp.concatenate([dst * NCODE + code,
                             jnp.full((pad,), N_NODES * NCODE, jnp.int32)]
                            ).reshape(TOTAL_CHUNKS, CH)
    idx_p = jnp.stack([src_p, dst_p, hix_p], axis=1)

    accs, hists = _sc_aggregate(h, idx_p)
    accs = accs[:, :N_NODES]

    etab = (emb1[:, None, :] + emb2[None, :, :]).reshape(NCODE, HID)
    slvec = (emb1[5] + emb2[0]).reshape(1, HID)
    hi0 = hists[:N_NODES * NCODE].reshape(N_NODES, NCODE)
    hi1 = hists[HBINS:HBINS + N_NODES * NCODE].reshape(N_NODES, NCODE)
    out = pl.pallas_call(
        _tc_mlp,
        grid=(_GRID,),
        in_specs=[
            pl.BlockSpec((_BLK, HID), lambda i: (i, 0)),
            pl.BlockSpec((_BLK, HID), lambda i: (i, 0)),
            pl.BlockSpec((_BLK, HID), lambda i: (i, 0)),
            pl.BlockSpec((_BLK, NCODE), lambda i: (i, 0)),
            pl.BlockSpec((_BLK, NCODE), lambda i: (i, 0)),
            pl.BlockSpec((NCODE, HID), lambda i: (0, 0)),
            pl.BlockSpec((1, HID), lambda i: (0, 0)),
            pl.BlockSpec((2 * HID, HID), lambda i: (0, 0)),
            pl.BlockSpec((1, 2 * HID), lambda i: (0, 0)),
            pl.BlockSpec((OUT, 2 * HID), lambda i: (0, 0)),
            pl.BlockSpec((1, OUT), lambda i: (0, 0)),
        ],
        out_specs=pl.BlockSpec((_BLK, OUT), lambda i: (i, 0)),
        out_shape=jax.ShapeDtypeStruct((N_NODES, OUT), f32),
    )(accs[0], accs[1], h, hi0, hi1, etab, slvec,
      W1, jnp.reshape(b1, (1, 2 * HID)), W2, jnp.reshape(b2, (1, OUT)))
    return out
